# Initial kernel scaffold; baseline (speedup 1.0000x reference)
import jax
import jax.numpy as jnp
from jax.experimental import pallas as pl


def kernel(x, edge_index, W_embed, b_embed, Wconv, att_src, att_dst, b_conv, W1, b1, W2, b2, ln_g, ln_b):
    raise NotImplementedError("write your pallas kernel here")


# speedup vs baseline: 33.8041x; 33.8041x over previous
import functools

import jax
import jax.numpy as jnp
from jax import lax
from jax.experimental import pallas as pl
from jax.experimental.pallas import tpu as pltpu
from jax.experimental.pallas import tpu_sc as plsc

_H = 4
_C = 64
_CHUNK = 64
_NS = 16
_NCORE = 2


def _embed_body(x_ref, w_ref, b_ref, o_ref):
    o_ref[...] = (
        jnp.dot(x_ref[...], w_ref[...], preferred_element_type=jnp.float32)
        + b_ref[...]
    ) * (64.0 ** 0.5)


def _prep_body(h_ref, w_ref, as_ref, ad_ref, h2_ref, alst_ref, aldt_ref,
               mx_ref):
    n = h_ref.shape[0]
    h256 = jnp.dot(h_ref[...], w_ref[...], preferred_element_type=jnp.float32)
    h2_ref[:n, :] = h256[:, :128]
    h2_ref[n:, :] = h256[:, 128:]
    row = lax.broadcasted_iota(jnp.int32, (_H * _C, _H), 0)
    col = lax.broadcasted_iota(jnp.int32, (_H * _C, _H), 1)
    sel = (row // _C) == col
    s_mat = jnp.where(sel, as_ref[...], 0.0)
    d_mat = jnp.where(sel, ad_ref[...], 0.0)
    dn = (((0,), (1,)), ((), ()))
    alst = lax.dot_general(s_mat, h256, dn, preferred_element_type=jnp.float32)
    aldt = lax.dot_general(d_mat, h256, dn, preferred_element_type=jnp.float32)
    alst_ref[...] = alst
    aldt_ref[...] = aldt
    als = jnp.dot(h256, s_mat, preferred_element_type=jnp.float32)
    mxk = jnp.max(als, axis=0, keepdims=True)
    mx_ref[...] = jnp.pad(mxk, ((0, 0), (0, 16 - _H)))


def _ssum_body(sp_ref, o_ref):
    o_ref[...] = jnp.sum(sp_ref[...], axis=1)


def _post_body(out_ref, s2_ref, h_ref, bc_ref, w1_ref, b1_ref, w2_ref, b2_ref,
               g_ref, bb_ref, o_ref):
    n = h_ref.shape[0]
    n_pad = out_ref.shape[0] // 2
    u = out_ref[:n, :]
    v = out_ref[n_pad:n_pad + n, :]
    r0 = 1.0 / (s2_ref[0, :, 0:1] + 1e-16)
    r1 = 1.0 / (s2_ref[0, :, 1:2] + 1e-16)
    r2 = 1.0 / (s2_ref[1, :, 0:1] + 1e-16)
    r3 = 1.0 / (s2_ref[1, :, 1:2] + 1e-16)
    o256 = jnp.concatenate(
        [u[:, 0:64] * r0, u[:, 64:128] * r1,
         v[:, 0:64] * r2, v[:, 64:128] * r3], axis=1) + bc_ref[...]
    y = jnp.dot(o256, w1_ref[...], preferred_element_type=jnp.float32) + b1_ref[...]
    y = y * 0.5 * (1.0 + lax.erf(y * (2.0 ** -0.5)))
    y = jnp.dot(y, w2_ref[...], preferred_element_type=jnp.float32) + b2_ref[...]
    mu = jnp.mean(y, axis=-1, keepdims=True)
    var = jnp.mean((y - mu) ** 2, axis=-1, keepdims=True)
    y = (y - mu) / jnp.sqrt(var + 1e-5) * g_ref[...] + bb_ref[...]
    o_ref[...] = h_ref[...] + y


def _tc_call(body, out_shape):
    return pl.pallas_call(body, out_shape=out_shape)


@functools.lru_cache(maxsize=None)
def _make_ex_kernel(n_nodes, e_tot, e_pad):
    edges_per_tile = e_pad // _NS
    n_chunks = edges_per_tile // _CHUNK
    s_words = 2 * n_nodes
    mesh = plsc.VectorSubcoreMesh(core_axis_name="c", subcore_axis_name="s")

    def body(alst_hbm, aldt_hbm, mx_hbm, src_hbm, dst_hbm, zflat_hbm,
             exm_hbm, souts_hbm,
             als_v, ald_v, mx_v, sidx_v, didx_v, exw0_v, exw1_v, spart_v):
        c = lax.axis_index("c")
        s = lax.axis_index("s")

        pltpu.sync_copy(alst_hbm.at[pl.ds(2 * c * n_nodes, 2 * n_nodes)], als_v)
        pltpu.sync_copy(aldt_hbm.at[pl.ds(2 * c * n_nodes, 2 * n_nodes)], ald_v)
        pltpu.sync_copy(mx_hbm, mx_v)
        pltpu.sync_copy(zflat_hbm, spart_v)

        li = lax.iota(jnp.int32, 16)
        h0 = 2 * c
        hi0 = jnp.full((16,), h0, jnp.int32)
        hi1 = jnp.full((16,), h0 + 1, jnp.int32)
        mrow = mx_v[0, pl.ds(0, 16)]
        mxv0 = mrow.at[hi0].get(mode="promise_in_bounds")
        mxv1 = mrow.at[hi1].get(mode="promise_in_bounds")
        zv = jnp.zeros((16,), jnp.int32)
        nv = jnp.full((16,), n_nodes, jnp.int32)
        lmasks = [li == l for l in range(16)]
        tile_base = s * edges_per_tile
        ex_base0 = h0 * e_pad + tile_base
        ex_base1 = (h0 + 1) * e_pad + tile_base

        def chunk_body(k, carry):
            base = tile_base + k * _CHUNK
            pltpu.sync_copy(src_hbm.at[pl.ds(base, _CHUNK)], sidx_v)
            pltpu.sync_copy(dst_hbm.at[pl.ds(base, _CHUNK)], didx_v)
            for g in range(_CHUNK // 16):
                sid = sidx_v[pl.ds(16 * g, 16)]
                did = didx_v[pl.ds(16 * g, 16)]
                eid = jnp.full((16,), base + 16 * g, jnp.int32) + li
                valid = eid < e_tot
                did2 = did * 2
                for hh, hoff, mxv, exw in ((0, zv, mxv0, exw0_v),
                                           (1, nv, mxv1, exw1_v)):
                    a_s = plsc.load_gather(als_v, [hoff + sid])
                    a_d = plsc.load_gather(ald_v, [hoff + did])
                    uu = a_s + a_d
                    e = jnp.maximum(uu, 0.2 * uu)
                    w = a_d + mxv
                    m = jnp.maximum(w, 0.2 * w)
                    ex = jnp.exp(e - m)
                    ex = jnp.where(valid, ex, 0.0)
                    exw[pl.ds(16 * g, 16)] = ex
                    fidx = did2 + hh
                    for l in range(16):
                        plsc.addupdate_scatter(spart_v, [fidx], ex,
                                               mask=lmasks[l])
            pltpu.sync_copy(exw0_v, exm_hbm.at[pl.ds(ex_base0 + k * _CHUNK,
                                                     _CHUNK)])
            pltpu.sync_copy(exw1_v, exm_hbm.at[pl.ds(ex_base1 + k * _CHUNK,
                                                     _CHUNK)])
            return carry

        lax.fori_loop(0, n_chunks, chunk_body, 0)
        pltpu.sync_copy(
            spart_v,
            souts_hbm.at[pl.ds((c * _NS + s) * s_words, s_words)])

    return pl.kernel(
        body,
        out_type=(
            jax.ShapeDtypeStruct((_H * e_pad,), jnp.float32),
            jax.ShapeDtypeStruct((2 * _NS * s_words,), jnp.float32),
        ),
        mesh=mesh,
        compiler_params=pltpu.CompilerParams(needs_layout_passes=False),
        scratch_types=[
            pltpu.VMEM((2 * n_nodes,), jnp.float32),
            pltpu.VMEM((2 * n_nodes,), jnp.float32),
            pltpu.VMEM((1, 16), jnp.float32),
            pltpu.VMEM((_CHUNK,), jnp.int32),
            pltpu.VMEM((_CHUNK,), jnp.int32),
            pltpu.VMEM((_CHUNK,), jnp.float32),
            pltpu.VMEM((_CHUNK,), jnp.float32),
            pltpu.VMEM((s_words,), jnp.float32),
        ],
    )


@functools.lru_cache(maxsize=None)
def _make_agg_kernel(n_nodes, n_pad, e_pad):
    edges_per_tile = e_pad // _NS
    n_chunks = edges_per_tile // _CHUNK
    rows_per_tile = n_pad // _NS
    mesh = plsc.VectorSubcoreMesh(core_axis_name="c", subcore_axis_name="s")

    def body(h2_hbm, exm_hbm, src_hbm, dst_hbm, zrows_hbm,
             out_hbm,
             sidx_v, sadj_v, didx_v, ex0_v, ex1_v, hrows_v, wrows_v, acc, sem):
        c = lax.axis_index("c")
        s = lax.axis_index("s")

        pltpu.sync_copy(zrows_hbm,
                        acc.at[pl.ds(s * rows_per_tile, rows_per_tile)])
        plsc.subcore_barrier()

        coff = jnp.full((16,), c * n_nodes, jnp.int32)
        h0 = 2 * c
        lanes = [jnp.full((16,), l, jnp.int32) for l in range(16)]
        tile_base = s * edges_per_tile
        ex_base0 = h0 * e_pad + tile_base
        ex_base1 = (h0 + 1) * e_pad + tile_base

        def chunk_body(k, carry):
            base = tile_base + k * _CHUNK
            pltpu.sync_copy(src_hbm.at[pl.ds(base, _CHUNK)], sidx_v)
            pltpu.sync_copy(dst_hbm.at[pl.ds(base, _CHUNK)], didx_v)
            for g in range(_CHUNK // 16):
                sadj_v[pl.ds(16 * g, 16)] = sidx_v[pl.ds(16 * g, 16)] + coff
            gd = pltpu.async_copy(h2_hbm.at[sadj_v], hrows_v, sem)
            pltpu.sync_copy(exm_hbm.at[pl.ds(ex_base0 + k * _CHUNK, _CHUNK)],
                            ex0_v)
            pltpu.sync_copy(exm_hbm.at[pl.ds(ex_base1 + k * _CHUNK, _CHUNK)],
                            ex1_v)
            gd.wait()

            for g in range(_CHUNK // 16):
                ex0v = ex0_v[pl.ds(16 * g, 16)]
                ex1v = ex1_v[pl.ds(16 * g, 16)]
                for l in range(16):
                    i = 16 * g + l
                    b0 = ex0v.at[lanes[l]].get(mode="promise_in_bounds")
                    b1 = ex1v.at[lanes[l]].get(mode="promise_in_bounds")
                    for j in range(8):
                        bb = b0 if j < 4 else b1
                        wrows_v[i, pl.ds(16 * j, 16)] = (
                            hrows_v[i, pl.ds(16 * j, 16)] * bb)

            pltpu.sync_copy(wrows_v, acc.at[didx_v], add=True)
            return carry

        lax.fori_loop(0, n_chunks, chunk_body, 0)
        plsc.subcore_barrier()
        pltpu.sync_copy(
            acc.at[pl.ds(s * rows_per_tile, rows_per_tile)],
            out_hbm.at[pl.ds(c * n_pad + s * rows_per_tile, rows_per_tile)])

    return pl.kernel(
        body,
        out_type=jax.ShapeDtypeStruct((2 * n_pad, 128), jnp.float32),
        mesh=mesh,
        compiler_params=pltpu.CompilerParams(needs_layout_passes=False),
        scratch_types=[
            pltpu.VMEM((_CHUNK,), jnp.int32),
            pltpu.VMEM((_CHUNK,), jnp.int32),
            pltpu.VMEM((_CHUNK,), jnp.int32),
            pltpu.VMEM((_CHUNK,), jnp.float32),
            pltpu.VMEM((_CHUNK,), jnp.float32),
            pltpu.VMEM((_CHUNK, 128), jnp.float32),
            pltpu.VMEM((_CHUNK, 128), jnp.float32),
            pltpu.VMEM_SHARED((n_pad, 128), jnp.float32),
            pltpu.SemaphoreType.DMA,
        ],
    )


def kernel(x, edge_index, W_embed, b_embed, Wconv, att_src, att_dst, b_conv,
           W1, b1, W2, b2, ln_g, ln_b):
    n = x.shape[0]
    e = edge_index.shape[1]
    e_tot = e + n
    unit = _NS * _CHUNK
    e_pad = ((e_tot + unit - 1) // unit) * unit
    n_unit = _NS * 8
    n_pad = ((n + n_unit - 1) // n_unit) * n_unit
    num_layers = Wconv.shape[0]
    hid = W_embed.shape[1]
    dff = W1.shape[2]
    s_words = 2 * n

    loop = jnp.arange(n, dtype=edge_index.dtype)
    pad = jnp.zeros((e_pad - e_tot,), dtype=edge_index.dtype)
    src = jnp.concatenate([edge_index[0], loop, pad])
    dst = jnp.concatenate([edge_index[1], loop, pad])
    zrows = jnp.zeros((n_pad // _NS, 128), jnp.float32)
    zflat = jnp.zeros((s_words,), jnp.float32)

    h = _tc_call(_embed_body, jax.ShapeDtypeStruct((n, hid), jnp.float32))(
        x, W_embed, b_embed.reshape(1, hid))

    ex_kernel = _make_ex_kernel(n, e_tot, e_pad)
    agg_kernel = _make_agg_kernel(n, n_pad, e_pad)
    prep = _tc_call(_prep_body, (
        jax.ShapeDtypeStruct((2 * n, 128), jnp.float32),
        jax.ShapeDtypeStruct((_H, n), jnp.float32),
        jax.ShapeDtypeStruct((_H, n), jnp.float32),
        jax.ShapeDtypeStruct((1, 16), jnp.float32),
    ))
    ssum = _tc_call(_ssum_body,
                    jax.ShapeDtypeStruct((2, s_words), jnp.float32))
    post = _tc_call(_post_body, jax.ShapeDtypeStruct((n, hid), jnp.float32))

    for i in range(num_layers):
        h2, alst, aldt, mx = prep(h, Wconv[i],
                                  att_src[i].reshape(_H * _C, 1),
                                  att_dst[i].reshape(_H * _C, 1))
        exm, souts = ex_kernel(alst.reshape(-1), aldt.reshape(-1), mx,
                               src, dst, zflat)
        out = agg_kernel(h2, exm, src, dst, zrows)
        s2 = ssum(souts.reshape(2, _NS, s_words)).reshape(2, n, 2)
        h = post(out, s2, h, b_conv[i].reshape(1, _H * _C),
                 W1[i], b1[i].reshape(1, dff), W2[i], b2[i].reshape(1, hid),
                 ln_g[i].reshape(1, hid), ln_b[i].reshape(1, hid))
    return h


# speedup vs baseline: 37.7677x; 1.1173x over previous
import functools

import jax
import jax.numpy as jnp
from jax import lax
from jax.experimental import pallas as pl
from jax.experimental.pallas import tpu as pltpu
from jax.experimental.pallas import tpu_sc as plsc

_H = 4
_C = 64
_CHUNK = 64
_NS = 16
_NCORE = 2


def _embed_body(x_ref, w_ref, b_ref, o_ref):
    o_ref[...] = (
        jnp.dot(x_ref[...], w_ref[...], preferred_element_type=jnp.float32)
        + b_ref[...]
    ) * (64.0 ** 0.5)


def _prep_body(h_ref, w_ref, as_ref, ad_ref, h2_ref, alst_ref, aldt_ref,
               mx_ref):
    n = h_ref.shape[0]
    h256 = jnp.dot(h_ref[...], w_ref[...], preferred_element_type=jnp.float32)
    h2_ref[:n, :] = h256[:, :128]
    h2_ref[n:, :] = h256[:, 128:]
    row = lax.broadcasted_iota(jnp.int32, (_H * _C, _H), 0)
    col = lax.broadcasted_iota(jnp.int32, (_H * _C, _H), 1)
    sel = (row // _C) == col
    s_mat = jnp.where(sel, as_ref[...], 0.0)
    d_mat = jnp.where(sel, ad_ref[...], 0.0)
    dn = (((0,), (1,)), ((), ()))
    alst = lax.dot_general(s_mat, h256, dn, preferred_element_type=jnp.float32)
    aldt = lax.dot_general(d_mat, h256, dn, preferred_element_type=jnp.float32)
    alst_ref[...] = alst
    aldt_ref[...] = aldt
    als = jnp.dot(h256, s_mat, preferred_element_type=jnp.float32)
    mxk = jnp.max(als, axis=0, keepdims=True)
    mx_ref[...] = jnp.pad(mxk, ((0, 0), (0, 16 - _H)))


def _post_body(out_ref, s2_ref, h_ref, bc_ref, w1_ref, b1_ref, w2_ref, b2_ref,
               g_ref, bb_ref, o_ref):
    n = h_ref.shape[0]
    n_pad = out_ref.shape[0] // 2
    u = out_ref[:n, :]
    v = out_ref[n_pad:n_pad + n, :]
    r0 = 1.0 / (s2_ref[0, :, 0:1] + 1e-16)
    r1 = 1.0 / (s2_ref[0, :, 1:2] + 1e-16)
    r2 = 1.0 / (s2_ref[1, :, 0:1] + 1e-16)
    r3 = 1.0 / (s2_ref[1, :, 1:2] + 1e-16)
    o256 = jnp.concatenate(
        [u[:, 0:64] * r0, u[:, 64:128] * r1,
         v[:, 0:64] * r2, v[:, 64:128] * r3], axis=1) + bc_ref[...]
    y = jnp.dot(o256, w1_ref[...], preferred_element_type=jnp.float32) + b1_ref[...]
    y = y * 0.5 * (1.0 + lax.erf(y * (2.0 ** -0.5)))
    y = jnp.dot(y, w2_ref[...], preferred_element_type=jnp.float32) + b2_ref[...]
    mu = jnp.mean(y, axis=-1, keepdims=True)
    var = jnp.mean((y - mu) ** 2, axis=-1, keepdims=True)
    y = (y - mu) / jnp.sqrt(var + 1e-5) * g_ref[...] + bb_ref[...]
    o_ref[...] = h_ref[...] + y


def _tc_call(body, out_shape):
    return pl.pallas_call(body, out_shape=out_shape)


@functools.lru_cache(maxsize=None)
def _make_ex_kernel(n_nodes, e_tot, e_pad):
    ck = 2 * _CHUNK
    edges_per_tile = e_pad // _NS
    n_chunks = edges_per_tile // ck
    s_words = 2 * n_nodes
    mesh = plsc.VectorSubcoreMesh(core_axis_name="c", subcore_axis_name="s")

    def body(alst_hbm, aldt_hbm, mx_hbm, src_hbm, dst_hbm, zflat_hbm,
             exm_hbm, souts_hbm,
             als_v, ald_v, mx_v, sidx_v, didx_v, exw0_v, exw1_v,
             fidx0_v, fidx1_v, stage_v, sacc):
        c = lax.axis_index("c")
        s = lax.axis_index("s")

        pltpu.sync_copy(alst_hbm.at[pl.ds(2 * c * n_nodes, 2 * n_nodes)], als_v)
        pltpu.sync_copy(aldt_hbm.at[pl.ds(2 * c * n_nodes, 2 * n_nodes)], ald_v)
        pltpu.sync_copy(mx_hbm, mx_v)

        @pl.when(s == 0)
        def _():
            pltpu.sync_copy(zflat_hbm, sacc)
        plsc.subcore_barrier()

        li = lax.iota(jnp.int32, 16)
        h0 = 2 * c
        hi0 = jnp.full((16,), h0, jnp.int32)
        hi1 = jnp.full((16,), h0 + 1, jnp.int32)
        mrow = mx_v[0, pl.ds(0, 16)]
        mxv0 = mrow.at[hi0].get(mode="promise_in_bounds")
        mxv1 = mrow.at[hi1].get(mode="promise_in_bounds")
        zv = jnp.zeros((16,), jnp.int32)
        nv = jnp.full((16,), n_nodes, jnp.int32)
        tile_base = s * edges_per_tile
        ex_base0 = h0 * e_pad + tile_base
        ex_base1 = (h0 + 1) * e_pad + tile_base

        def chunk_body(k, carry):
            base = tile_base + k * ck
            pltpu.sync_copy(src_hbm.at[pl.ds(base, ck)], sidx_v)
            pltpu.sync_copy(dst_hbm.at[pl.ds(base, ck)], didx_v)
            for g in range(ck // 16):
                sid = sidx_v[pl.ds(16 * g, 16)]
                did = didx_v[pl.ds(16 * g, 16)]
                eid = jnp.full((16,), base + 16 * g, jnp.int32) + li
                valid = eid < e_tot
                did2 = did * 2
                for hh, hoff, mxv, exw, fxw in (
                        (0, zv, mxv0, exw0_v, fidx0_v),
                        (1, nv, mxv1, exw1_v, fidx1_v)):
                    a_s = plsc.load_gather(als_v, [hoff + sid])
                    a_d = plsc.load_gather(ald_v, [hoff + did])
                    uu = a_s + a_d
                    e = jnp.maximum(uu, 0.2 * uu)
                    w = a_d + mxv
                    m = jnp.maximum(w, 0.2 * w)
                    ex = jnp.exp(e - m)
                    ex = jnp.where(valid, ex, 0.0)
                    exw[pl.ds(16 * g, 16)] = ex
                    fxw[pl.ds(16 * g, 16)] = did2 + hh
            pltpu.sync_copy(exw0_v, sacc.at[fidx0_v], add=True)
            pltpu.sync_copy(exw1_v, sacc.at[fidx1_v], add=True)
            pltpu.sync_copy(exw0_v, exm_hbm.at[pl.ds(ex_base0 + k * ck, ck)])
            pltpu.sync_copy(exw1_v, exm_hbm.at[pl.ds(ex_base1 + k * ck, ck)])
            return carry

        lax.fori_loop(0, n_chunks, chunk_body, 0)
        plsc.subcore_barrier()

        @pl.when(s == 0)
        def _():
            pltpu.sync_copy(sacc, stage_v)
            pltpu.sync_copy(stage_v, souts_hbm.at[pl.ds(c * s_words, s_words)])

    return pl.kernel(
        body,
        out_type=(
            jax.ShapeDtypeStruct((_H * e_pad,), jnp.float32),
            jax.ShapeDtypeStruct((2 * s_words,), jnp.float32),
        ),
        mesh=mesh,
        compiler_params=pltpu.CompilerParams(needs_layout_passes=False),
        scratch_types=[
            pltpu.VMEM((2 * n_nodes,), jnp.float32),
            pltpu.VMEM((2 * n_nodes,), jnp.float32),
            pltpu.VMEM((1, 16), jnp.float32),
            pltpu.VMEM((ck,), jnp.int32),
            pltpu.VMEM((ck,), jnp.int32),
            pltpu.VMEM((ck,), jnp.float32),
            pltpu.VMEM((ck,), jnp.float32),
            pltpu.VMEM((ck,), jnp.int32),
            pltpu.VMEM((ck,), jnp.int32),
            pltpu.VMEM((s_words,), jnp.float32),
            pltpu.VMEM_SHARED((s_words,), jnp.float32),
        ],
    )


@functools.lru_cache(maxsize=None)
def _make_agg_kernel(n_nodes, n_pad, e_pad):
    edges_per_tile = e_pad // _NS
    n_chunks = edges_per_tile // _CHUNK
    rows_per_tile = n_pad // _NS
    mesh = plsc.VectorSubcoreMesh(core_axis_name="c", subcore_axis_name="s")

    def body(h2_hbm, exm_hbm, src_hbm, dst_hbm, zrows_hbm,
             out_hbm,
             sidx_v, sadj_v, didx_v, ex0_v, ex1_v, hrows_v, wrows_v, acc, sem):
        c = lax.axis_index("c")
        s = lax.axis_index("s")

        pltpu.sync_copy(zrows_hbm,
                        acc.at[pl.ds(s * rows_per_tile, rows_per_tile)])
        plsc.subcore_barrier()

        coff = jnp.full((16,), c * n_nodes, jnp.int32)
        h0 = 2 * c
        lanes = [jnp.full((16,), l, jnp.int32) for l in range(16)]
        tile_base = s * edges_per_tile
        ex_base0 = h0 * e_pad + tile_base
        ex_base1 = (h0 + 1) * e_pad + tile_base

        def chunk_body(k, carry):
            base = tile_base + k * _CHUNK
            pltpu.sync_copy(src_hbm.at[pl.ds(base, _CHUNK)], sidx_v)
            pltpu.sync_copy(dst_hbm.at[pl.ds(base, _CHUNK)], didx_v)
            for g in range(_CHUNK // 16):
                sadj_v[pl.ds(16 * g, 16)] = sidx_v[pl.ds(16 * g, 16)] + coff
            gd = pltpu.async_copy(h2_hbm.at[sadj_v], hrows_v, sem)
            pltpu.sync_copy(exm_hbm.at[pl.ds(ex_base0 + k * _CHUNK, _CHUNK)],
                            ex0_v)
            pltpu.sync_copy(exm_hbm.at[pl.ds(ex_base1 + k * _CHUNK, _CHUNK)],
                            ex1_v)
            gd.wait()

            for g in range(_CHUNK // 16):
                ex0v = ex0_v[pl.ds(16 * g, 16)]
                ex1v = ex1_v[pl.ds(16 * g, 16)]
                for l in range(16):
                    i = 16 * g + l
                    b0 = ex0v.at[lanes[l]].get(mode="promise_in_bounds")
                    b1 = ex1v.at[lanes[l]].get(mode="promise_in_bounds")
                    for j in range(8):
                        bb = b0 if j < 4 else b1
                        wrows_v[i, pl.ds(16 * j, 16)] = (
                            hrows_v[i, pl.ds(16 * j, 16)] * bb)

            pltpu.sync_copy(wrows_v, acc.at[didx_v], add=True)
            return carry

        lax.fori_loop(0, n_chunks, chunk_body, 0)
        plsc.subcore_barrier()
        pltpu.sync_copy(
            acc.at[pl.ds(s * rows_per_tile, rows_per_tile)],
            out_hbm.at[pl.ds(c * n_pad + s * rows_per_tile, rows_per_tile)])

    return pl.kernel(
        body,
        out_type=jax.ShapeDtypeStruct((2 * n_pad, 128), jnp.float32),
        mesh=mesh,
        compiler_params=pltpu.CompilerParams(needs_layout_passes=False),
        scratch_types=[
            pltpu.VMEM((_CHUNK,), jnp.int32),
            pltpu.VMEM((_CHUNK,), jnp.int32),
            pltpu.VMEM((_CHUNK,), jnp.int32),
            pltpu.VMEM((_CHUNK,), jnp.float32),
            pltpu.VMEM((_CHUNK,), jnp.float32),
            pltpu.VMEM((_CHUNK, 128), jnp.float32),
            pltpu.VMEM((_CHUNK, 128), jnp.float32),
            pltpu.VMEM_SHARED((n_pad, 128), jnp.float32),
            pltpu.SemaphoreType.DMA,
        ],
    )


def kernel(x, edge_index, W_embed, b_embed, Wconv, att_src, att_dst, b_conv,
           W1, b1, W2, b2, ln_g, ln_b):
    n = x.shape[0]
    e = edge_index.shape[1]
    e_tot = e + n
    unit = _NS * 2 * _CHUNK
    e_pad = ((e_tot + unit - 1) // unit) * unit
    n_unit = _NS * 8
    n_pad = ((n + n_unit - 1) // n_unit) * n_unit
    num_layers = Wconv.shape[0]
    hid = W_embed.shape[1]
    dff = W1.shape[2]
    s_words = 2 * n

    loop = jnp.arange(n, dtype=edge_index.dtype)
    pad = jnp.zeros((e_pad - e_tot,), dtype=edge_index.dtype)
    src = jnp.concatenate([edge_index[0], loop, pad])
    dst = jnp.concatenate([edge_index[1], loop, pad])
    zrows = jnp.zeros((n_pad // _NS, 128), jnp.float32)
    zflat = jnp.zeros((s_words,), jnp.float32)

    h = _tc_call(_embed_body, jax.ShapeDtypeStruct((n, hid), jnp.float32))(
        x, W_embed, b_embed.reshape(1, hid))

    ex_kernel = _make_ex_kernel(n, e_tot, e_pad)
    agg_kernel = _make_agg_kernel(n, n_pad, e_pad)
    prep = _tc_call(_prep_body, (
        jax.ShapeDtypeStruct((2 * n, 128), jnp.float32),
        jax.ShapeDtypeStruct((_H, n), jnp.float32),
        jax.ShapeDtypeStruct((_H, n), jnp.float32),
        jax.ShapeDtypeStruct((1, 16), jnp.float32),
    ))
    post = _tc_call(_post_body, jax.ShapeDtypeStruct((n, hid), jnp.float32))

    for i in range(num_layers):
        h2, alst, aldt, mx = prep(h, Wconv[i],
                                  att_src[i].reshape(_H * _C, 1),
                                  att_dst[i].reshape(_H * _C, 1))
        exm, souts = ex_kernel(alst.reshape(-1), aldt.reshape(-1), mx,
                               src, dst, zflat)
        out = agg_kernel(h2, exm, src, dst, zrows)
        s2 = souts.reshape(2, n, 2)
        h = post(out, s2, h, b_conv[i].reshape(1, _H * _C),
                 W1[i], b1[i].reshape(1, dff), W2[i], b2[i].reshape(1, hid),
                 ln_g[i].reshape(1, hid), ln_b[i].reshape(1, hid))
    return h


# speedup vs baseline: 52.8082x; 1.3982x over previous
import functools

import jax
import jax.numpy as jnp
from jax import lax
from jax.experimental import pallas as pl
from jax.experimental.pallas import tpu as pltpu
from jax.experimental.pallas import tpu_sc as plsc

_H = 4
_C = 64
_CHUNK = 64
_NS = 16
_NCORE = 2


def _embed_body(x_ref, w_ref, b_ref, o_ref):
    o_ref[...] = (
        jnp.dot(x_ref[...], w_ref[...], preferred_element_type=jnp.float32)
        + b_ref[...]
    ) * (64.0 ** 0.5)


def _prep_body(h_ref, w_ref, as_ref, ad_ref, h2_ref, alst_ref, aldt_ref,
               mx_ref):
    n = h_ref.shape[0]
    h256 = jnp.dot(h_ref[...], w_ref[...], preferred_element_type=jnp.float32)
    h2_ref[:n, :] = h256[:, :128]
    h2_ref[n:, :] = h256[:, 128:]
    row = lax.broadcasted_iota(jnp.int32, (_H * _C, _H), 0)
    col = lax.broadcasted_iota(jnp.int32, (_H * _C, _H), 1)
    sel = (row // _C) == col
    s_mat = jnp.where(sel, as_ref[...], 0.0)
    d_mat = jnp.where(sel, ad_ref[...], 0.0)
    dn = (((0,), (1,)), ((), ()))
    alst = lax.dot_general(s_mat, h256, dn, preferred_element_type=jnp.float32)
    aldt = lax.dot_general(d_mat, h256, dn, preferred_element_type=jnp.float32)
    alst_ref[...] = alst
    aldt_ref[...] = aldt
    als = jnp.dot(h256, s_mat, preferred_element_type=jnp.float32)
    mxk = jnp.max(als, axis=0, keepdims=True)
    mx_ref[...] = jnp.pad(mxk, ((0, 0), (0, 16 - _H)))


def _post_body(out_ref, s2_ref, h_ref, bc_ref, w1_ref, b1_ref, w2_ref, b2_ref,
               g_ref, bb_ref, o_ref):
    n = h_ref.shape[0]
    n_pad = out_ref.shape[0] // 2
    u = out_ref[:n, :]
    v = out_ref[n_pad:n_pad + n, :]
    r0 = 1.0 / (s2_ref[0, :, 0:1] + 1e-16)
    r1 = 1.0 / (s2_ref[0, :, 1:2] + 1e-16)
    r2 = 1.0 / (s2_ref[1, :, 0:1] + 1e-16)
    r3 = 1.0 / (s2_ref[1, :, 1:2] + 1e-16)
    o256 = jnp.concatenate(
        [u[:, 0:64] * r0, u[:, 64:128] * r1,
         v[:, 0:64] * r2, v[:, 64:128] * r3], axis=1) + bc_ref[...]
    y = jnp.dot(o256, w1_ref[...], preferred_element_type=jnp.float32) + b1_ref[...]
    y = y * 0.5 * (1.0 + lax.erf(y * (2.0 ** -0.5)))
    y = jnp.dot(y, w2_ref[...], preferred_element_type=jnp.float32) + b2_ref[...]
    mu = jnp.mean(y, axis=-1, keepdims=True)
    var = jnp.mean((y - mu) ** 2, axis=-1, keepdims=True)
    y = (y - mu) / jnp.sqrt(var + 1e-5) * g_ref[...] + bb_ref[...]
    o_ref[...] = h_ref[...] + y


def _tc_call(body, out_shape):
    return pl.pallas_call(body, out_shape=out_shape)


@functools.lru_cache(maxsize=None)
def _make_ex_kernel(n_nodes, e_tot, e_pad):
    ck = 2 * _CHUNK
    edges_per_tile = e_pad // _NS
    n_chunks = edges_per_tile // ck
    s_words = 2 * n_nodes
    mesh = plsc.VectorSubcoreMesh(core_axis_name="c", subcore_axis_name="s")

    def body(alst_hbm, aldt_hbm, mx_hbm, src_hbm, dst_hbm, zflat_hbm,
             exm_hbm, souts_hbm,
             als_v, ald_v, mx_v, sidx_v, didx_v, exw0_v, exw1_v,
             fidx0_v, fidx1_v, stage_v, sacc):
        c = lax.axis_index("c")
        s = lax.axis_index("s")

        pltpu.sync_copy(alst_hbm.at[pl.ds(2 * c * n_nodes, 2 * n_nodes)], als_v)
        pltpu.sync_copy(aldt_hbm.at[pl.ds(2 * c * n_nodes, 2 * n_nodes)], ald_v)
        pltpu.sync_copy(mx_hbm, mx_v)

        @pl.when(s == 0)
        def _():
            pltpu.sync_copy(zflat_hbm, sacc)
        plsc.subcore_barrier()

        li = lax.iota(jnp.int32, 16)
        h0 = 2 * c
        hi0 = jnp.full((16,), h0, jnp.int32)
        hi1 = jnp.full((16,), h0 + 1, jnp.int32)
        mrow = mx_v[0, pl.ds(0, 16)]
        mxv0 = mrow.at[hi0].get(mode="promise_in_bounds")
        mxv1 = mrow.at[hi1].get(mode="promise_in_bounds")
        zv = jnp.zeros((16,), jnp.int32)
        nv = jnp.full((16,), n_nodes, jnp.int32)
        tile_base = s * edges_per_tile
        ex_base0 = h0 * e_pad + tile_base
        ex_base1 = (h0 + 1) * e_pad + tile_base

        def chunk_body(k, carry):
            base = tile_base + k * ck
            pltpu.sync_copy(src_hbm.at[pl.ds(base, ck)], sidx_v)
            pltpu.sync_copy(dst_hbm.at[pl.ds(base, ck)], didx_v)
            for g in range(ck // 16):
                sid = sidx_v[pl.ds(16 * g, 16)]
                did = didx_v[pl.ds(16 * g, 16)]
                eid = jnp.full((16,), base + 16 * g, jnp.int32) + li
                valid = eid < e_tot
                did2 = did * 2
                for hh, hoff, mxv, exw, fxw in (
                        (0, zv, mxv0, exw0_v, fidx0_v),
                        (1, nv, mxv1, exw1_v, fidx1_v)):
                    a_s = plsc.load_gather(als_v, [hoff + sid])
                    a_d = plsc.load_gather(ald_v, [hoff + did])
                    uu = a_s + a_d
                    e = jnp.maximum(uu, 0.2 * uu)
                    w = a_d + mxv
                    m = jnp.maximum(w, 0.2 * w)
                    ex = jnp.exp(e - m)
                    ex = jnp.where(valid, ex, 0.0)
                    exw[pl.ds(16 * g, 16)] = ex
                    fxw[pl.ds(16 * g, 16)] = did2 + hh
            pltpu.sync_copy(exw0_v, sacc.at[fidx0_v], add=True)
            pltpu.sync_copy(exw1_v, sacc.at[fidx1_v], add=True)
            pltpu.sync_copy(exw0_v, exm_hbm.at[pl.ds(ex_base0 + k * ck, ck)])
            pltpu.sync_copy(exw1_v, exm_hbm.at[pl.ds(ex_base1 + k * ck, ck)])
            return carry

        lax.fori_loop(0, n_chunks, chunk_body, 0)
        plsc.subcore_barrier()

        @pl.when(s == 0)
        def _():
            pltpu.sync_copy(sacc, stage_v)
            pltpu.sync_copy(stage_v, souts_hbm.at[pl.ds(c * s_words, s_words)])

    return pl.kernel(
        body,
        out_type=(
            jax.ShapeDtypeStruct((_H * e_pad,), jnp.float32),
            jax.ShapeDtypeStruct((2 * s_words,), jnp.float32),
        ),
        mesh=mesh,
        compiler_params=pltpu.CompilerParams(needs_layout_passes=False),
        scratch_types=[
            pltpu.VMEM((2 * n_nodes,), jnp.float32),
            pltpu.VMEM((2 * n_nodes,), jnp.float32),
            pltpu.VMEM((1, 16), jnp.float32),
            pltpu.VMEM((ck,), jnp.int32),
            pltpu.VMEM((ck,), jnp.int32),
            pltpu.VMEM((ck,), jnp.float32),
            pltpu.VMEM((ck,), jnp.float32),
            pltpu.VMEM((ck,), jnp.int32),
            pltpu.VMEM((ck,), jnp.int32),
            pltpu.VMEM((s_words,), jnp.float32),
            pltpu.VMEM_SHARED((s_words,), jnp.float32),
        ],
    )


@functools.lru_cache(maxsize=None)
def _make_agg_kernel(n_nodes, n_pad, e_pad):
    nb = 8
    blk = nb * _CHUNK
    edges_per_tile = e_pad // _NS
    n_blocks = edges_per_tile // blk
    rows_per_tile = n_pad // _NS
    mesh = plsc.VectorSubcoreMesh(core_axis_name="c", subcore_axis_name="s")

    def body(h2_hbm, exm_hbm, src2_hbm, dst2_hbm, zrows_hbm,
             out_hbm,
             srcb_v, dstb_v, ex0_v, ex1_v,
             sadj0_v, sadj1_v, sadj2_v, sadj3_v,
             didx0_v, didx1_v, didx2_v, didx3_v,
             hrows0_v, hrows1_v, wrows0_v, wrows1_v, acc,
             bsem, gsem0, gsem1, ssem0, ssem1):
        c = lax.axis_index("c")
        s = lax.axis_index("s")

        pltpu.sync_copy(zrows_hbm,
                        acc.at[pl.ds(s * rows_per_tile, rows_per_tile)])
        plsc.subcore_barrier()

        coff = jnp.full((16,), c * n_nodes, jnp.int32)
        h0 = 2 * c
        lanes = [jnp.full((16,), l, jnp.int32) for l in range(16)]
        tile_base = s * edges_per_tile
        ex_base0 = h0 * e_pad + tile_base
        ex_base1 = (h0 + 1) * e_pad + tile_base
        sadj = (sadj0_v, sadj1_v, sadj2_v, sadj3_v)
        didx = (didx0_v, didx1_v, didx2_v, didx3_v)
        hrows = (hrows0_v, hrows1_v)
        wrows = (wrows0_v, wrows1_v)
        gsem = (gsem0, gsem1)
        ssem = (ssem0, ssem1)

        def fill_sadj(b, k):
            for g in range(_CHUNK // 16):
                sadj[b][pl.ds(16 * g, 16)] = (
                    srcb_v[k, 0, pl.ds(16 * g, 16)] + coff)
                didx[b][pl.ds(16 * g, 16)] = dstb_v[k, 0, pl.ds(16 * g, 16)]

        def block_body(kb, carry):
            base = tile_base + kb * blk
            row = base // _CHUNK
            d1 = pltpu.async_copy(src2_hbm.at[pl.ds(row, nb)], srcb_v, bsem)
            d2 = pltpu.async_copy(dst2_hbm.at[pl.ds(row, nb)], dstb_v, bsem)
            d3 = pltpu.async_copy(exm_hbm.at[pl.ds(ex_base0 + kb * blk, blk)],
                                  ex0_v, bsem)
            d4 = pltpu.async_copy(exm_hbm.at[pl.ds(ex_base1 + kb * blk, blk)],
                                  ex1_v, bsem)
            d1.wait(); d2.wait(); d3.wait(); d4.wait()

            fill_sadj(0, 0)
            gd = [pltpu.async_copy(h2_hbm.at[sadj[0]], hrows[0], gsem[0]),
                  None]
            sd = [None, None]
            for k in range(nb):
                p = k % 2
                q = 1 - p
                if k + 1 < nb:
                    fill_sadj((k + 1) % 4, k + 1)
                    gd[q] = pltpu.async_copy(h2_hbm.at[sadj[(k + 1) % 4]],
                                             hrows[q], gsem[q])
                gd[p].wait()
                if sd[p] is not None:
                    sd[p].wait()
                for g in range(_CHUNK // 16):
                    ex0v = ex0_v[pl.ds(k * _CHUNK + 16 * g, 16)]
                    ex1v = ex1_v[pl.ds(k * _CHUNK + 16 * g, 16)]
                    for l in range(16):
                        i = 16 * g + l
                        b0 = ex0v.at[lanes[l]].get(mode="promise_in_bounds")
                        b1 = ex1v.at[lanes[l]].get(mode="promise_in_bounds")
                        for j in range(8):
                            bb = b0 if j < 4 else b1
                            wrows[p][i, pl.ds(16 * j, 16)] = (
                                hrows[p][i, pl.ds(16 * j, 16)] * bb)
                sd[p] = pltpu.async_copy(wrows[p], acc.at[didx[k % 4]],
                                         ssem[p], add=True)
            if sd[0] is not None:
                sd[0].wait()
            if sd[1] is not None:
                sd[1].wait()
            return carry

        lax.fori_loop(0, n_blocks, block_body, 0)
        plsc.subcore_barrier()
        pltpu.sync_copy(
            acc.at[pl.ds(s * rows_per_tile, rows_per_tile)],
            out_hbm.at[pl.ds(c * n_pad + s * rows_per_tile, rows_per_tile)])

    return pl.kernel(
        body,
        out_type=jax.ShapeDtypeStruct((2 * n_pad, 128), jnp.float32),
        mesh=mesh,
        compiler_params=pltpu.CompilerParams(needs_layout_passes=False),
        scratch_types=[
            pltpu.VMEM((nb, 1, _CHUNK), jnp.int32),
            pltpu.VMEM((nb, 1, _CHUNK), jnp.int32),
            pltpu.VMEM((blk,), jnp.float32),
            pltpu.VMEM((blk,), jnp.float32),
            pltpu.VMEM((_CHUNK,), jnp.int32),
            pltpu.VMEM((_CHUNK,), jnp.int32),
            pltpu.VMEM((_CHUNK,), jnp.int32),
            pltpu.VMEM((_CHUNK,), jnp.int32),
            pltpu.VMEM((_CHUNK,), jnp.int32),
            pltpu.VMEM((_CHUNK,), jnp.int32),
            pltpu.VMEM((_CHUNK,), jnp.int32),
            pltpu.VMEM((_CHUNK,), jnp.int32),
            pltpu.VMEM((_CHUNK, 128), jnp.float32),
            pltpu.VMEM((_CHUNK, 128), jnp.float32),
            pltpu.VMEM((_CHUNK, 128), jnp.float32),
            pltpu.VMEM((_CHUNK, 128), jnp.float32),
            pltpu.VMEM_SHARED((n_pad, 128), jnp.float32),
            pltpu.SemaphoreType.DMA,
            pltpu.SemaphoreType.DMA,
            pltpu.SemaphoreType.DMA,
            pltpu.SemaphoreType.DMA,
            pltpu.SemaphoreType.DMA,
        ],
    )


def kernel(x, edge_index, W_embed, b_embed, Wconv, att_src, att_dst, b_conv,
           W1, b1, W2, b2, ln_g, ln_b):
    n = x.shape[0]
    e = edge_index.shape[1]
    e_tot = e + n
    unit = _NS * 8 * _CHUNK
    e_pad = ((e_tot + unit - 1) // unit) * unit
    n_unit = _NS * 8
    n_pad = ((n + n_unit - 1) // n_unit) * n_unit
    num_layers = Wconv.shape[0]
    hid = W_embed.shape[1]
    dff = W1.shape[2]
    s_words = 2 * n

    loop = jnp.arange(n, dtype=edge_index.dtype)
    pad = jnp.zeros((e_pad - e_tot,), dtype=edge_index.dtype)
    src = jnp.concatenate([edge_index[0], loop, pad])
    dst = jnp.concatenate([edge_index[1], loop, pad])
    zrows = jnp.zeros((n_pad // _NS, 128), jnp.float32)
    zflat = jnp.zeros((s_words,), jnp.float32)

    h = _tc_call(_embed_body, jax.ShapeDtypeStruct((n, hid), jnp.float32))(
        x, W_embed, b_embed.reshape(1, hid))

    ex_kernel = _make_ex_kernel(n, e_tot, e_pad)
    agg_kernel = _make_agg_kernel(n, n_pad, e_pad)
    prep = _tc_call(_prep_body, (
        jax.ShapeDtypeStruct((2 * n, 128), jnp.float32),
        jax.ShapeDtypeStruct((_H, n), jnp.float32),
        jax.ShapeDtypeStruct((_H, n), jnp.float32),
        jax.ShapeDtypeStruct((1, 16), jnp.float32),
    ))
    post = _tc_call(_post_body, jax.ShapeDtypeStruct((n, hid), jnp.float32))

    for i in range(num_layers):
        h2, alst, aldt, mx = prep(h, Wconv[i],
                                  att_src[i].reshape(_H * _C, 1),
                                  att_dst[i].reshape(_H * _C, 1))
        exm, souts = ex_kernel(alst.reshape(-1), aldt.reshape(-1), mx,
                               src, dst, zflat)
        out = agg_kernel(h2, exm, src.reshape(-1, 1, _CHUNK),
                         dst.reshape(-1, 1, _CHUNK), zrows)
        s2 = souts.reshape(2, n, 2)
        h = post(out, s2, h, b_conv[i].reshape(1, _H * _C),
                 W1[i], b1[i].reshape(1, dff), W2[i], b2[i].reshape(1, hid),
                 ln_g[i].reshape(1, hid), ln_b[i].reshape(1, hid))
    return h
